# baseline (speedup 1.0000x reference)
import dataclasses
import functools

import jax
import jax.numpy as jnp
from jax import lax
from jax.experimental import pallas as pl
from jax.experimental.pallas import tpu as pltpu
from jax.experimental.pallas import tpu_sc as plsc

N = 4096
E = 131072
HID = 128
NW = 32
PLANE = N * N

NCHUNK = 2
EC = E // NCHUNK

_EBLK = 2048
_GSPLIT = 8


def _gather_body(rel_hbm, force_hbm, t_hbm, s_hbm, out_hbm, ts_v, idx_v,
                 rel_stage, cbuf2, cbuf3, force_v, sem, *, ew, base):
    cid = lax.axis_index("c")
    sid = lax.axis_index("s")
    wid = sid * 2 + cid
    pltpu.sync_copy(t_hbm.at[pl.ds(base + wid * ew, ew)], ts_v.at[pl.ds(0, ew)])
    pltpu.sync_copy(s_hbm.at[pl.ds(base + wid * ew, ew)],
                    ts_v.at[pl.ds(ew, ew)])
    pltpu.sync_copy(force_hbm, force_v)

    @pl.loop(0, ew // 16)
    def _(g):
        tr = ts_v[pl.ds(g * 16, 16)]
        sr = ts_v[pl.ds(ew + g * 16, 16)]
        off = ((tr >> 3) * 32768 + (sr >> 7) * 1024
               + (tr & 7) * 128 + (sr & 127))
        idx_v[pl.ds(g * 16, 16)] = off
        idx_v[pl.ds(ew + g * 16, 16)] = off + PLANE
        idx_v[pl.ds(2 * ew + g * 16, 16)] = off + 2 * PLANE

    sub = ew // _GSPLIT
    handles = []
    for k in range(3):
        for j in range(_GSPLIT):
            o = k * ew + j * sub
            handles.append(pltpu.async_copy(
                rel_hbm.at[idx_v.at[pl.ds(o, sub)]],
                rel_stage.at[pl.ds(o, sub)], sem))

    @pl.loop(0, ew // 16)
    def _(g):
        cgo = (g // 8) * 1024 + (g % 8) * 16
        tr = ts_v[pl.ds(g * 16, 16)]
        sr = ts_v[pl.ds(ew + g * 16, 16)]
        cbuf2[pl.ds(cgo + 3 * 128, 16)] = plsc.load_gather(force_v, [tr])
        cbuf2[pl.ds(cgo + 4 * 128, 16)] = plsc.load_gather(force_v, [tr + N])
        cbuf2[pl.ds(cgo + 5 * 128, 16)] = plsc.load_gather(force_v,
                                                           [tr + 2 * N])
        cbuf2[pl.ds(cgo + 6 * 128, 16)] = plsc.load_gather(force_v, [sr])
        cbuf2[pl.ds(cgo + 7 * 128, 16)] = plsc.load_gather(force_v, [sr + N])
        cbuf3[pl.ds(cgo, 16)] = plsc.load_gather(force_v, [sr + 2 * N])

    for h in handles:
        h.wait()

    @pl.loop(0, ew // 16)
    def _(g):
        cgo = (g // 8) * 1024 + (g % 8) * 16
        for c in range(3):
            cbuf2[pl.ds(cgo + c * 128, 16)] = rel_stage[
                pl.ds(c * ew + g * 16, 16)]

    h1 = pltpu.async_copy(cbuf2, out_hbm.at[pl.ds(wid * 8 * ew, 8 * ew)], sem)
    h2 = pltpu.async_copy(
        cbuf3, out_hbm.at[pl.ds((NW + wid) * 8 * ew, 8 * ew)], sem)
    h1.wait()
    h2.wait()


def _mlp_body(med_ref, x_ref, w1t_ref, beff_ref, w2t_ref, b2_ref, out_ref):
    a = x_ref[...]
    d2 = a[0:1] * a[0:1] + a[1:2] * a[1:2] + a[2:3] * a[2:3]
    dist = jnp.maximum(jnp.sqrt(d2), 1e-8)
    rsh = dist - med_ref[0]
    r2 = rsh * rsh
    r4 = r2 * r2
    x = jnp.concatenate([a[0:9], dist, r2, r4], axis=0)
    h = jnp.tanh(
        lax.dot_general(w1t_ref[...], x, (((1,), (0,)), ((), ())),
                        preferred_element_type=jnp.float32) + beff_ref[...])
    v = lax.dot_general(w2t_ref[...], h, (((1,), (0,)), ((), ())),
                        preferred_element_type=jnp.float32) + b2_ref[...]
    out_ref[...] = jnp.concatenate(
        [v, jnp.zeros((5, v.shape[1]), jnp.float32)], axis=0)


def _scatter_body(vel_hbm, t_hbm, zeros_hbm, out_hbm, vel_v, vs, tv,
                  sidx_v0, sidx_v1, sidx_v2, acc, sem, *, ew, base):
    cid = lax.axis_index("c")
    sid = lax.axis_index("s")
    wid = sid * 2 + cid
    pltpu.sync_copy(vel_hbm.at[pl.ds(wid * 8 * ew, 8 * ew)], vel_v)
    pltpu.sync_copy(t_hbm.at[pl.ds(base + wid * ew, ew)], tv)

    @pl.loop(0, ew // 16)
    def _(g):
        cgo = (g // 8) * 1024 + (g % 8) * 16
        for c, sv in enumerate((sidx_v0, sidx_v1, sidx_v2)):
            vs[pl.ds(c * ew + g * 16, 16)] = vel_v[pl.ds(cgo + c * 128, 16)]
            sv[pl.ds(g * 16, 16)] = tv[pl.ds(g * 16, 16)] * 3 + c

    @pl.when(sid == 0)
    def _():
        pltpu.sync_copy(zeros_hbm, acc)

    plsc.subcore_barrier()
    hs = [pltpu.async_copy(vs.at[pl.ds(c * ew, ew)], acc.at[sv], sem,
                           add=True)
          for c, sv in enumerate((sidx_v0, sidx_v1, sidx_v2))]
    for hcopy in hs:
        hcopy.wait()
    plsc.subcore_barrier()

    @pl.when(sid < 12)
    def _():
        pltpu.sync_copy(acc.at[pl.ds(sid * 1024, 1024)],
                        out_hbm.at[pl.ds(cid * 12288 + sid * 1024, 1024)])


def kernel(rel_vecs, target_indices, source_indices, force, viscosity,
           W1, b1, W2, b2, median, contact_distance):
    t = target_indices.astype(jnp.int32)
    s = source_indices.astype(jnp.int32)

    rel_flat = (rel_vecs.transpose(2, 0, 1)
                .reshape(3, N // 8, 8, N // 128, 128)
                .transpose(0, 1, 3, 2, 4)
                .reshape(3 * PLANE))
    force_pl = force.T.reshape(3 * N)

    mu = jnp.asarray(viscosity, jnp.float32)
    w1x = jnp.concatenate(
        [W1[0:3], W1[7:10], W1[10:13], (W1[3] + W1[6])[None],
         W1[4:5], W1[5:6]], axis=0)
    w1t = w1x.T
    beff = (b1 + mu * W1[13] - contact_distance * W1[6]).reshape(HID, 1)
    w2t = W2.T
    b2c = b2.reshape(3, 1)
    med = jnp.asarray(median, jnp.float32).reshape(1)
    zeros = jnp.zeros((3 * N,), jnp.float32)

    mesh = plsc.VectorSubcoreMesh(core_axis_name="c", subcore_axis_name="s")
    cp = pltpu.CompilerParams()
    if "needs_layout_passes" in pltpu.CompilerParams.__dataclass_fields__:
        cp = dataclasses.replace(cp, needs_layout_passes=False)

    ew = EC // NW

    def gather_k(h):
        return pl.kernel(
            functools.partial(_gather_body, ew=ew, base=h * EC), mesh=mesh,
            out_type=jax.ShapeDtypeStruct((16 * EC,), jnp.float32),
            scratch_types=[pltpu.VMEM((2 * ew,), jnp.int32),
                           pltpu.VMEM((3 * ew,), jnp.int32),
                           pltpu.VMEM((3 * ew,), jnp.float32),
                           pltpu.VMEM((8 * ew,), jnp.float32),
                           pltpu.VMEM((8 * ew,), jnp.float32),
                           pltpu.VMEM((3 * N,), jnp.float32),
                           pltpu.SemaphoreType.DMA],
            compiler_params=cp,
        )

    def scatter_k(h):
        return pl.kernel(
            functools.partial(_scatter_body, ew=ew, base=h * EC), mesh=mesh,
            out_type=jax.ShapeDtypeStruct((2 * 3 * N,), jnp.float32),
            scratch_types=[pltpu.VMEM((8 * ew,), jnp.float32),
                           pltpu.VMEM((3 * ew,), jnp.float32),
                           pltpu.VMEM((ew,), jnp.int32),
                           pltpu.VMEM((ew,), jnp.int32),
                           pltpu.VMEM((ew,), jnp.int32),
                           pltpu.VMEM((ew,), jnp.int32),
                           pltpu.VMEM_SHARED((3 * N,), jnp.float32),
                           pltpu.SemaphoreType.DMA],
            compiler_params=cp,
        )

    mlp_k = pl.pallas_call(
        _mlp_body,
        grid=(EC // _EBLK,),
        in_specs=[pl.BlockSpec(memory_space=pltpu.SMEM),
                  pl.BlockSpec((16, _EBLK), lambda i: (0, i)),
                  pl.BlockSpec((HID, 12), lambda i: (0, 0)),
                  pl.BlockSpec((HID, 1), lambda i: (0, 0)),
                  pl.BlockSpec((3, HID), lambda i: (0, 0)),
                  pl.BlockSpec((3, 1), lambda i: (0, 0))],
        out_specs=pl.BlockSpec((8, _EBLK), lambda i: (0, i)),
        out_shape=jax.ShapeDtypeStruct((8, EC), jnp.float32),
    )

    partial_sums = []
    for h in range(NCHUNK):
        planes = gather_k(h)(rel_flat, force_pl, t, s)
        x_tc = (planes.reshape(2, NW * (ew // 128), 8, 128)
                .transpose(0, 2, 1, 3).reshape(16, EC))
        vel = mlp_k(med, x_tc, w1t, beff, w2t, b2c)
        vel1d = (vel.reshape(8, NW * (ew // 128), 128)
                 .transpose(1, 0, 2).reshape(8 * EC))
        partial_sums.append(scatter_k(h)(vel1d, t, zeros))

    p = sum(partial_sums).reshape(2, 3 * N)
    return (p[0] + p[1]).reshape(N, 3)


# speedup vs baseline: 1.1776x; 1.1776x over previous
import dataclasses
import functools

import jax
import jax.numpy as jnp
from jax import lax
from jax.experimental import pallas as pl
from jax.experimental.pallas import tpu as pltpu
from jax.experimental.pallas import tpu_sc as plsc

N = 4096
E = 131072
HID = 128
NW = 32
PLANE = N * N

NCHUNK = 2
EC = E // NCHUNK

_EBLK = 4096
_GSPLIT = 4


def _gather_body(rel_hbm, force_hbm, t_hbm, s_hbm, out_hbm, ts_v, idx_v,
                 rel_stage, cbuf2, cbuf3, force_v, sem, *, ew, base):
    cid = lax.axis_index("c")
    sid = lax.axis_index("s")
    wid = sid * 2 + cid
    pltpu.sync_copy(t_hbm.at[pl.ds(base + wid * ew, ew)], ts_v.at[pl.ds(0, ew)])
    pltpu.sync_copy(s_hbm.at[pl.ds(base + wid * ew, ew)],
                    ts_v.at[pl.ds(ew, ew)])
    pltpu.sync_copy(force_hbm, force_v)

    @pl.loop(0, ew // 16)
    def _(g):
        tr = ts_v[pl.ds(g * 16, 16)]
        sr = ts_v[pl.ds(ew + g * 16, 16)]
        off = ((tr >> 3) * 32768 + (sr >> 7) * 1024
               + (tr & 7) * 128 + (sr & 127))
        idx_v[pl.ds(g * 16, 16)] = off
        idx_v[pl.ds(ew + g * 16, 16)] = off + PLANE
        idx_v[pl.ds(2 * ew + g * 16, 16)] = off + 2 * PLANE

    sub = ew // _GSPLIT
    handles = []
    for k in range(3):
        for j in range(_GSPLIT):
            o = k * ew + j * sub
            handles.append(pltpu.async_copy(
                rel_hbm.at[idx_v.at[pl.ds(o, sub)]],
                rel_stage.at[pl.ds(o, sub)], sem))

    @pl.loop(0, ew // 16)
    def _(g):
        cgo = (g // 8) * 1024 + (g % 8) * 16
        tr = ts_v[pl.ds(g * 16, 16)]
        sr = ts_v[pl.ds(ew + g * 16, 16)]
        cbuf2[pl.ds(cgo + 3 * 128, 16)] = plsc.load_gather(force_v, [tr])
        cbuf2[pl.ds(cgo + 4 * 128, 16)] = plsc.load_gather(force_v, [tr + N])
        cbuf2[pl.ds(cgo + 5 * 128, 16)] = plsc.load_gather(force_v,
                                                           [tr + 2 * N])
        cbuf2[pl.ds(cgo + 6 * 128, 16)] = plsc.load_gather(force_v, [sr])
        cbuf2[pl.ds(cgo + 7 * 128, 16)] = plsc.load_gather(force_v, [sr + N])
        cbuf3[pl.ds(cgo, 16)] = plsc.load_gather(force_v, [sr + 2 * N])

    for h in handles:
        h.wait()

    @pl.loop(0, ew // 16)
    def _(g):
        cgo = (g // 8) * 1024 + (g % 8) * 16
        for c in range(3):
            cbuf2[pl.ds(cgo + c * 128, 16)] = rel_stage[
                pl.ds(c * ew + g * 16, 16)]

    h1 = pltpu.async_copy(cbuf2, out_hbm.at[pl.ds(wid * 8 * ew, 8 * ew)], sem)
    h2 = pltpu.async_copy(
        cbuf3, out_hbm.at[pl.ds((NW + wid) * 8 * ew, 8 * ew)], sem)
    h1.wait()
    h2.wait()


def _mlp_body(med_ref, x_ref, w1t_ref, beff_ref, w2t_ref, b2_ref, out_ref):
    a = x_ref[...]
    d2 = a[0:1] * a[0:1] + a[1:2] * a[1:2] + a[2:3] * a[2:3]
    dist = jnp.maximum(jnp.sqrt(d2), 1e-8)
    rsh = dist - med_ref[0]
    r2 = rsh * rsh
    r4 = r2 * r2
    x = jnp.concatenate([a[0:9], dist, r2, r4], axis=0)
    h = jnp.tanh(
        lax.dot_general(w1t_ref[...], x, (((1,), (0,)), ((), ())),
                        preferred_element_type=jnp.float32) + beff_ref[...])
    v = lax.dot_general(w2t_ref[...], h, (((1,), (0,)), ((), ())),
                        preferred_element_type=jnp.float32) + b2_ref[...]
    out_ref[...] = jnp.concatenate(
        [v, jnp.zeros((5, v.shape[1]), jnp.float32)], axis=0)


def _scatter_body(vel_hbm, t_hbm, zeros_hbm, out_hbm, vel_v, vs, tv,
                  sidx_v0, sidx_v1, sidx_v2, acc, sem, *, ew, base):
    cid = lax.axis_index("c")
    sid = lax.axis_index("s")
    wid = sid * 2 + cid
    pltpu.sync_copy(vel_hbm.at[pl.ds(wid * 8 * ew, 8 * ew)], vel_v)
    pltpu.sync_copy(t_hbm.at[pl.ds(base + wid * ew, ew)], tv)

    @pl.loop(0, ew // 16)
    def _(g):
        cgo = (g // 8) * 1024 + (g % 8) * 16
        for c, sv in enumerate((sidx_v0, sidx_v1, sidx_v2)):
            vs[pl.ds(c * ew + g * 16, 16)] = vel_v[pl.ds(cgo + c * 128, 16)]
            sv[pl.ds(g * 16, 16)] = tv[pl.ds(g * 16, 16)] * 3 + c

    @pl.when(sid == 0)
    def _():
        pltpu.sync_copy(zeros_hbm, acc)

    plsc.subcore_barrier()
    hs = [pltpu.async_copy(vs.at[pl.ds(c * ew, ew)], acc.at[sv], sem,
                           add=True)
          for c, sv in enumerate((sidx_v0, sidx_v1, sidx_v2))]
    for hcopy in hs:
        hcopy.wait()
    plsc.subcore_barrier()

    @pl.when(sid < 12)
    def _():
        pltpu.sync_copy(acc.at[pl.ds(sid * 1024, 1024)],
                        out_hbm.at[pl.ds(cid * 12288 + sid * 1024, 1024)])


def kernel(rel_vecs, target_indices, source_indices, force, viscosity,
           W1, b1, W2, b2, median, contact_distance):
    t = target_indices.astype(jnp.int32)
    s = source_indices.astype(jnp.int32)

    rel_flat = (rel_vecs.transpose(2, 0, 1)
                .reshape(3, N // 8, 8, N // 128, 128)
                .transpose(0, 1, 3, 2, 4)
                .reshape(3 * PLANE))
    force_pl = force.T.reshape(3 * N)

    mu = jnp.asarray(viscosity, jnp.float32)
    w1x = jnp.concatenate(
        [W1[0:3], W1[7:10], W1[10:13], (W1[3] + W1[6])[None],
         W1[4:5], W1[5:6]], axis=0)
    w1t = w1x.T
    beff = (b1 + mu * W1[13] - contact_distance * W1[6]).reshape(HID, 1)
    w2t = W2.T
    b2c = b2.reshape(3, 1)
    med = jnp.asarray(median, jnp.float32).reshape(1)
    zeros = jnp.zeros((3 * N,), jnp.float32)

    mesh = plsc.VectorSubcoreMesh(core_axis_name="c", subcore_axis_name="s")
    cp = pltpu.CompilerParams()
    if "needs_layout_passes" in pltpu.CompilerParams.__dataclass_fields__:
        cp = dataclasses.replace(cp, needs_layout_passes=False)

    ew = EC // NW

    def gather_k(h):
        return pl.kernel(
            functools.partial(_gather_body, ew=ew, base=h * EC), mesh=mesh,
            out_type=jax.ShapeDtypeStruct((16 * EC,), jnp.float32),
            scratch_types=[pltpu.VMEM((2 * ew,), jnp.int32),
                           pltpu.VMEM((3 * ew,), jnp.int32),
                           pltpu.VMEM((3 * ew,), jnp.float32),
                           pltpu.VMEM((8 * ew,), jnp.float32),
                           pltpu.VMEM((8 * ew,), jnp.float32),
                           pltpu.VMEM((3 * N,), jnp.float32),
                           pltpu.SemaphoreType.DMA],
            compiler_params=cp,
        )

    def scatter_k(h):
        return pl.kernel(
            functools.partial(_scatter_body, ew=ew, base=h * EC), mesh=mesh,
            out_type=jax.ShapeDtypeStruct((2 * 3 * N,), jnp.float32),
            scratch_types=[pltpu.VMEM((8 * ew,), jnp.float32),
                           pltpu.VMEM((3 * ew,), jnp.float32),
                           pltpu.VMEM((ew,), jnp.int32),
                           pltpu.VMEM((ew,), jnp.int32),
                           pltpu.VMEM((ew,), jnp.int32),
                           pltpu.VMEM((ew,), jnp.int32),
                           pltpu.VMEM_SHARED((3 * N,), jnp.float32),
                           pltpu.SemaphoreType.DMA],
            compiler_params=cp,
        )

    mlp_k = pl.pallas_call(
        _mlp_body,
        grid=(EC // _EBLK,),
        in_specs=[pl.BlockSpec(memory_space=pltpu.SMEM),
                  pl.BlockSpec((16, _EBLK), lambda i: (0, i)),
                  pl.BlockSpec((HID, 12), lambda i: (0, 0)),
                  pl.BlockSpec((HID, 1), lambda i: (0, 0)),
                  pl.BlockSpec((3, HID), lambda i: (0, 0)),
                  pl.BlockSpec((3, 1), lambda i: (0, 0))],
        out_specs=pl.BlockSpec((8, _EBLK), lambda i: (0, i)),
        out_shape=jax.ShapeDtypeStruct((8, EC), jnp.float32),
    )

    partial_sums = []
    for h in range(NCHUNK):
        planes = gather_k(h)(rel_flat, force_pl, t, s)
        x_tc = (planes.reshape(2, NW * (ew // 128), 8, 128)
                .transpose(0, 2, 1, 3).reshape(16, EC))
        vel = mlp_k(med, x_tc, w1t, beff, w2t, b2c)
        vel1d = (vel.reshape(8, NW * (ew // 128), 128)
                 .transpose(1, 0, 2).reshape(8 * EC))
        partial_sums.append(scatter_k(h)(vel1d, t, zeros))

    p = sum(partial_sums).reshape(2, 3 * N)
    return (p[0] + p[1]).reshape(N, 3)


# speedup vs baseline: 1.1837x; 1.0051x over previous
import dataclasses
import functools

import jax
import jax.numpy as jnp
from jax import lax
from jax.experimental import pallas as pl
from jax.experimental.pallas import tpu as pltpu
from jax.experimental.pallas import tpu_sc as plsc

N = 4096
E = 131072
HID = 128
NW = 32
PLANE = N * N

NCHUNK = 2
EC = E // NCHUNK

_EBLK = 8192
_GSPLIT = 4


def _gather_body(rel_hbm, force_hbm, t_hbm, s_hbm, out_hbm, ts_v, idx_v,
                 rel_stage, cbuf2, cbuf3, force_v, sem, *, ew, base):
    cid = lax.axis_index("c")
    sid = lax.axis_index("s")
    wid = sid * 2 + cid
    pltpu.sync_copy(t_hbm.at[pl.ds(base + wid * ew, ew)], ts_v.at[pl.ds(0, ew)])
    pltpu.sync_copy(s_hbm.at[pl.ds(base + wid * ew, ew)],
                    ts_v.at[pl.ds(ew, ew)])
    pltpu.sync_copy(force_hbm, force_v)

    @pl.loop(0, ew // 16)
    def _(g):
        tr = ts_v[pl.ds(g * 16, 16)]
        sr = ts_v[pl.ds(ew + g * 16, 16)]
        off = ((tr >> 3) * 32768 + (sr >> 7) * 1024
               + (tr & 7) * 128 + (sr & 127))
        idx_v[pl.ds(g * 16, 16)] = off
        idx_v[pl.ds(ew + g * 16, 16)] = off + PLANE
        idx_v[pl.ds(2 * ew + g * 16, 16)] = off + 2 * PLANE

    sub = ew // _GSPLIT
    handles = []
    for k in range(3):
        for j in range(_GSPLIT):
            o = k * ew + j * sub
            handles.append(pltpu.async_copy(
                rel_hbm.at[idx_v.at[pl.ds(o, sub)]],
                rel_stage.at[pl.ds(o, sub)], sem))

    @pl.loop(0, ew // 16)
    def _(g):
        cgo = (g // 8) * 1024 + (g % 8) * 16
        tr = ts_v[pl.ds(g * 16, 16)]
        sr = ts_v[pl.ds(ew + g * 16, 16)]
        cbuf2[pl.ds(cgo + 3 * 128, 16)] = plsc.load_gather(force_v, [tr])
        cbuf2[pl.ds(cgo + 4 * 128, 16)] = plsc.load_gather(force_v, [tr + N])
        cbuf2[pl.ds(cgo + 5 * 128, 16)] = plsc.load_gather(force_v,
                                                           [tr + 2 * N])
        cbuf2[pl.ds(cgo + 6 * 128, 16)] = plsc.load_gather(force_v, [sr])
        cbuf2[pl.ds(cgo + 7 * 128, 16)] = plsc.load_gather(force_v, [sr + N])
        cbuf3[pl.ds(cgo, 16)] = plsc.load_gather(force_v, [sr + 2 * N])

    for h in handles:
        h.wait()

    @pl.loop(0, ew // 16)
    def _(g):
        cgo = (g // 8) * 1024 + (g % 8) * 16
        for c in range(3):
            cbuf2[pl.ds(cgo + c * 128, 16)] = rel_stage[
                pl.ds(c * ew + g * 16, 16)]

    h1 = pltpu.async_copy(cbuf2, out_hbm.at[pl.ds(wid * 8 * ew, 8 * ew)], sem)
    h2 = pltpu.async_copy(
        cbuf3, out_hbm.at[pl.ds((NW + wid) * 8 * ew, 8 * ew)], sem)
    h1.wait()
    h2.wait()


def _mlp_body(med_ref, x_ref, w1t_ref, beff_ref, w2t_ref, b2_ref, out_ref):
    a = x_ref[...]
    d2 = a[0:1] * a[0:1] + a[1:2] * a[1:2] + a[2:3] * a[2:3]
    dist = jnp.maximum(jnp.sqrt(d2), 1e-8)
    rsh = dist - med_ref[0]
    r2 = rsh * rsh
    r4 = r2 * r2
    x = jnp.concatenate([a[0:9], dist, r2, r4], axis=0)
    h = jnp.tanh(
        lax.dot_general(w1t_ref[...], x, (((1,), (0,)), ((), ())),
                        preferred_element_type=jnp.float32) + beff_ref[...])
    v = lax.dot_general(w2t_ref[...], h, (((1,), (0,)), ((), ())),
                        preferred_element_type=jnp.float32) + b2_ref[...]
    out_ref[...] = jnp.concatenate(
        [v, jnp.zeros((5, v.shape[1]), jnp.float32)], axis=0)


def _scatter_body(vel_hbm, t_hbm, zeros_hbm, out_hbm, vel_v, vs, tv,
                  sidx_v0, sidx_v1, sidx_v2, acc, sem, *, ew, base):
    cid = lax.axis_index("c")
    sid = lax.axis_index("s")
    wid = sid * 2 + cid
    pltpu.sync_copy(vel_hbm.at[pl.ds(wid * 8 * ew, 8 * ew)], vel_v)
    pltpu.sync_copy(t_hbm.at[pl.ds(base + wid * ew, ew)], tv)

    @pl.loop(0, ew // 16)
    def _(g):
        cgo = (g // 8) * 1024 + (g % 8) * 16
        for c, sv in enumerate((sidx_v0, sidx_v1, sidx_v2)):
            vs[pl.ds(c * ew + g * 16, 16)] = vel_v[pl.ds(cgo + c * 128, 16)]
            sv[pl.ds(g * 16, 16)] = tv[pl.ds(g * 16, 16)] * 3 + c

    @pl.when(sid == 0)
    def _():
        pltpu.sync_copy(zeros_hbm, acc)

    plsc.subcore_barrier()
    hs = [pltpu.async_copy(vs.at[pl.ds(c * ew, ew)], acc.at[sv], sem,
                           add=True)
          for c, sv in enumerate((sidx_v0, sidx_v1, sidx_v2))]
    for hcopy in hs:
        hcopy.wait()
    plsc.subcore_barrier()

    @pl.when(sid < 12)
    def _():
        pltpu.sync_copy(acc.at[pl.ds(sid * 1024, 1024)],
                        out_hbm.at[pl.ds(cid * 12288 + sid * 1024, 1024)])


def kernel(rel_vecs, target_indices, source_indices, force, viscosity,
           W1, b1, W2, b2, median, contact_distance):
    t = target_indices.astype(jnp.int32)
    s = source_indices.astype(jnp.int32)

    rel_flat = (rel_vecs.transpose(2, 0, 1)
                .reshape(3, N // 8, 8, N // 128, 128)
                .transpose(0, 1, 3, 2, 4)
                .reshape(3 * PLANE))
    force_pl = force.T.reshape(3 * N)

    mu = jnp.asarray(viscosity, jnp.float32)
    w1x = jnp.concatenate(
        [W1[0:3], W1[7:10], W1[10:13], (W1[3] + W1[6])[None],
         W1[4:5], W1[5:6]], axis=0)
    w1t = w1x.T
    beff = (b1 + mu * W1[13] - contact_distance * W1[6]).reshape(HID, 1)
    w2t = W2.T
    b2c = b2.reshape(3, 1)
    med = jnp.asarray(median, jnp.float32).reshape(1)
    zeros = jnp.zeros((3 * N,), jnp.float32)

    mesh = plsc.VectorSubcoreMesh(core_axis_name="c", subcore_axis_name="s")
    cp = pltpu.CompilerParams()
    if "needs_layout_passes" in pltpu.CompilerParams.__dataclass_fields__:
        cp = dataclasses.replace(cp, needs_layout_passes=False)

    ew = EC // NW

    def gather_k(h):
        return pl.kernel(
            functools.partial(_gather_body, ew=ew, base=h * EC), mesh=mesh,
            out_type=jax.ShapeDtypeStruct((16 * EC,), jnp.float32),
            scratch_types=[pltpu.VMEM((2 * ew,), jnp.int32),
                           pltpu.VMEM((3 * ew,), jnp.int32),
                           pltpu.VMEM((3 * ew,), jnp.float32),
                           pltpu.VMEM((8 * ew,), jnp.float32),
                           pltpu.VMEM((8 * ew,), jnp.float32),
                           pltpu.VMEM((3 * N,), jnp.float32),
                           pltpu.SemaphoreType.DMA],
            compiler_params=cp,
        )

    def scatter_k(h):
        return pl.kernel(
            functools.partial(_scatter_body, ew=ew, base=h * EC), mesh=mesh,
            out_type=jax.ShapeDtypeStruct((2 * 3 * N,), jnp.float32),
            scratch_types=[pltpu.VMEM((8 * ew,), jnp.float32),
                           pltpu.VMEM((3 * ew,), jnp.float32),
                           pltpu.VMEM((ew,), jnp.int32),
                           pltpu.VMEM((ew,), jnp.int32),
                           pltpu.VMEM((ew,), jnp.int32),
                           pltpu.VMEM((ew,), jnp.int32),
                           pltpu.VMEM_SHARED((3 * N,), jnp.float32),
                           pltpu.SemaphoreType.DMA],
            compiler_params=cp,
        )

    mlp_k = pl.pallas_call(
        _mlp_body,
        grid=(EC // _EBLK,),
        in_specs=[pl.BlockSpec(memory_space=pltpu.SMEM),
                  pl.BlockSpec((16, _EBLK), lambda i: (0, i)),
                  pl.BlockSpec((HID, 12), lambda i: (0, 0)),
                  pl.BlockSpec((HID, 1), lambda i: (0, 0)),
                  pl.BlockSpec((3, HID), lambda i: (0, 0)),
                  pl.BlockSpec((3, 1), lambda i: (0, 0))],
        out_specs=pl.BlockSpec((8, _EBLK), lambda i: (0, i)),
        out_shape=jax.ShapeDtypeStruct((8, EC), jnp.float32),
    )

    partial_sums = []
    for h in range(NCHUNK):
        planes = gather_k(h)(rel_flat, force_pl, t, s)
        x_tc = (planes.reshape(2, NW * (ew // 128), 8, 128)
                .transpose(0, 2, 1, 3).reshape(16, EC))
        vel = mlp_k(med, x_tc, w1t, beff, w2t, b2c)
        vel1d = (vel.reshape(8, NW * (ew // 128), 128)
                 .transpose(1, 0, 2).reshape(8 * EC))
        partial_sums.append(scatter_k(h)(vel1d, t, zeros))

    p = sum(partial_sums).reshape(2, 3 * N)
    return (p[0] + p[1]).reshape(N, 3)
